# Initial kernel scaffold; baseline (speedup 1.0000x reference)
import jax
import jax.numpy as jnp
from jax.experimental import pallas as pl


def kernel(data, imps, rows, cols):
    raise NotImplementedError("write your pallas kernel here")


# speedup vs baseline: 47.4812x; 47.4812x over previous
import functools

import jax
import jax.numpy as jnp
from jax import lax
from jax.experimental import pallas as pl
from jax.experimental.pallas import tpu as pltpu
from jax.experimental.pallas import tpu_sc as plsc

NC = 2
NS = 16
NW = NC * NS
L = 16

D = 64
CHUNK_ROWS = 625
CHUNK_WORDS = CHUNK_ROWS * D
K = 4096


def _scalar_at(buf, j):
    vec = plsc.load_gather(buf, [jnp.full((L,), j, jnp.int32)])
    return lax.reduce_max(vec, axes=(0,))


def _make_impl(n_rows, n_imps, starts_pad):
    total = n_rows * D
    num_chunks = n_rows // CHUNK_ROWS
    cpt = num_chunks // NW
    nlast = max(0, ((n_imps - K + 7) // 8) * 8)
    mesh = plsc.VectorSubcoreMesh(core_axis_name="c", subcore_axis_name="s")

    @functools.partial(
        pl.kernel,
        out_type=jax.ShapeDtypeStruct((total,), jnp.float32),
        mesh=mesh,
        compiler_params=pltpu.CompilerParams(needs_layout_passes=False),
        scratch_types=[
            pltpu.VMEM((CHUNK_WORDS,), jnp.float32),
            pltpu.VMEM((K,), jnp.int32),
            pltpu.VMEM((K,), jnp.int32),
            pltpu.VMEM((K,), jnp.float32),
            pltpu.VMEM((starts_pad,), jnp.int32),
            pltpu.SemaphoreType.DMA,
        ],
    )
    def impl(data_h, imps_h, rows_h, cols_h, starts_h, out_h,
             chunk_v, rows_v, cols_v, vals_v, starts_v, sem):
        wid = lax.axis_index("s") * NC + lax.axis_index("c")
        pltpu.sync_copy(starts_h, starts_v)

        def chunk_body(jl, carry):
            j = wid * cpt + jl
            s = _scalar_at(starts_v, j)
            e = _scalar_at(starts_v, j + 1)
            chunk_off = pl.multiple_of(j * CHUNK_WORDS, 8)
            pltpu.sync_copy(data_h.at[pl.ds(chunk_off, CHUNK_WORDS)], chunk_v)

            s0 = (s // 8) * 8
            nblk = (e - s0 + K - 1) // K

            def blk_body(b, carry2):
                cur = s0 + b * K
                off = pl.multiple_of(jnp.minimum(cur, nlast), 8)
                c1 = pltpu.async_copy(rows_h.at[pl.ds(off, K)], rows_v, sem)
                c2 = pltpu.async_copy(cols_h.at[pl.ds(off, K)], cols_v, sem)
                c3 = pltpu.async_copy(imps_h.at[pl.ds(off, K)], vals_v, sem)
                c1.wait()
                c2.wait()
                c3.wait()
                hi = jnp.minimum(e, off + K)
                ng = (hi - off + (L - 1)) // L

                def g_body(g, carry3):
                    sl = pl.ds(g * L, L)
                    r = rows_v[sl]
                    c = cols_v[sl]
                    v = vals_v[sl]
                    p = off + g * L + lax.iota(jnp.int32, L)
                    m = (p >= s) & (p < e)
                    lidx = r * D + c - chunk_off
                    plsc.store_scatter(chunk_v, [lidx], v, mask=m)
                    return carry3

                lax.fori_loop(0, ng, g_body, 0)
                return carry2

            lax.fori_loop(0, nblk, blk_body, 0)
            pltpu.sync_copy(chunk_v, out_h.at[pl.ds(chunk_off, CHUNK_WORDS)])
            return carry

        lax.fori_loop(0, cpt, chunk_body, 0)

    return impl


def kernel(data, imps, rows, cols):
    n_rows, d = data.shape
    assert d == D and n_rows % (CHUNK_ROWS * NW) == 0
    n_imps = imps.shape[0]
    if 0 < n_imps < K:
        pad = K - n_imps
        imps = jnp.pad(imps, (0, pad))
        rows = jnp.pad(rows, (0, pad))
        cols = jnp.pad(cols, (0, pad))

    num_chunks = n_rows // CHUNK_ROWS
    bounds = jnp.arange(0, n_rows + 1, CHUNK_ROWS, dtype=jnp.int32)
    starts = jnp.searchsorted(rows[:n_imps], bounds, side="left").astype(jnp.int32)
    starts_pad = ((num_chunks + 1 + 7) // 8) * 8
    starts = jnp.concatenate(
        [starts, jnp.full((starts_pad - num_chunks - 1,), n_imps, jnp.int32)])

    impl = _make_impl(n_rows, n_imps, starts_pad)
    out = impl(data.reshape(-1), imps, rows, cols, starts)
    return out.reshape(n_rows, D)


# speedup vs baseline: 55.3561x; 1.1659x over previous
import functools

import jax
import jax.numpy as jnp
from jax import lax
from jax.experimental import pallas as pl
from jax.experimental.pallas import tpu as pltpu
from jax.experimental.pallas import tpu_sc as plsc

NC = 2
NS = 16
NW = NC * NS
L = 16

D = 64
CHUNK_ROWS = 625
CHUNK_WORDS = CHUNK_ROWS * D
K = 12288


def _scalar_at(buf, j):
    vec = plsc.load_gather(buf, [jnp.full((L,), j, jnp.int32)])
    return lax.reduce_max(vec, axes=(0,))


def _make_impl(n_rows, n_imps, starts_pad):
    total = n_rows * D
    num_chunks = n_rows // CHUNK_ROWS
    cpt = num_chunks // NW
    assert cpt % 2 == 0
    nlast = max(0, ((max(n_imps, K) - K + 7) // 8) * 8)
    mesh = plsc.VectorSubcoreMesh(core_axis_name="c", subcore_axis_name="s")

    @functools.partial(
        pl.kernel,
        out_type=jax.ShapeDtypeStruct((total,), jnp.float32),
        mesh=mesh,
        compiler_params=pltpu.CompilerParams(needs_layout_passes=False),
        scratch_types=[
            pltpu.VMEM((CHUNK_WORDS,), jnp.float32),
            pltpu.VMEM((CHUNK_WORDS,), jnp.float32),
            pltpu.VMEM((K,), jnp.int32),
            pltpu.VMEM((K,), jnp.int32),
            pltpu.VMEM((K,), jnp.float32),
            pltpu.VMEM((starts_pad,), jnp.int32),
            pltpu.SemaphoreType.DMA,
            pltpu.SemaphoreType.DMA,
            pltpu.SemaphoreType.DMA,
            pltpu.SemaphoreType.DMA,
            pltpu.SemaphoreType.DMA,
        ],
    )
    def impl(data_h, imps_h, rows_h, cols_h, starts_h, out_h,
             chunk_v0, chunk_v1, rows_v, cols_v, vals_v, starts_v,
             sem_in0, sem_in1, sem_out0, sem_out1, sem_st):
        chunks = (chunk_v0, chunk_v1)
        wid = lax.axis_index("s") * NC + lax.axis_index("c")
        cbase = wid * cpt
        sems_in = (sem_in0, sem_in1)
        sems_out = (sem_out0, sem_out1)
        pltpu.sync_copy(starts_h, starts_v)
        iota = lax.iota(jnp.int32, L)

        def in_copy(c, b):
            off = pl.multiple_of((cbase + c) * CHUNK_WORDS, 8)
            return pltpu.make_async_copy(
                data_h.at[pl.ds(off, CHUNK_WORDS)], chunks[b], sems_in[b])

        def out_copy(c, b):
            off = pl.multiple_of((cbase + c) * CHUNK_WORDS, 8)
            return pltpu.make_async_copy(
                chunks[b], out_h.at[pl.ds(off, CHUNK_WORDS)], sems_out[b])

        def stream_copies(off):
            off = pl.multiple_of(off, 8)
            return (
                pltpu.make_async_copy(rows_h.at[pl.ds(off, K)], rows_v, sem_st),
                pltpu.make_async_copy(cols_h.at[pl.ds(off, K)], cols_v, sem_st),
                pltpu.make_async_copy(imps_h.at[pl.ds(off, K)], vals_v, sem_st),
            )

        def block_off(s0, b2):
            return pl.multiple_of(jnp.minimum(s0 + b2 * K, nlast), 8)

        def scatter_block(b, cw_base, off, s, e):
            chunk_ref = chunks[b]
            hi = jnp.minimum(e, off + K)
            ng = (hi - off + L - 1) // L
            glo = jnp.minimum((jnp.maximum(s - off, 0) + L - 1) // L, ng)
            ghi = jnp.maximum((hi - off) // L, glo)

            def masked(g, carry):
                sl = pl.ds(g * L, L)
                r = rows_v[sl]
                cc = cols_v[sl]
                v = vals_v[sl]
                p = off + g * L + iota
                m = (p >= s) & (p < e)
                plsc.store_scatter(chunk_ref, [r * D + cc - cw_base], v, mask=m)
                return carry

            lax.fori_loop(0, glo, masked, 0)

            @plsc.parallel_loop(glo, ghi, unroll=4)
            def _(g):
                sl = pl.ds(g * L, L)
                r = rows_v[sl]
                cc = cols_v[sl]
                v = vals_v[sl]
                plsc.store_scatter(chunk_ref, [r * D + cc - cw_base], v)

            lax.fori_loop(ghi, ng, masked, 0)

        s0_first = _scalar_at(starts_v, cbase)
        e_first = _scalar_at(starts_v, cbase + 1)
        in_copy(0, 0).start()
        for cp in stream_copies(block_off((s0_first // 8) * 8, 0)):
            cp.start()

        def outer(jo, carry):
            s, e = carry
            for b in (0, 1):
                c = 2 * jo + b
                nb = 1 - b

                @pl.when(c >= 1)
                def _():
                    out_copy(c - 1, nb).wait()

                @pl.when(c + 1 < cpt)
                def _():
                    in_copy(c + 1, nb).start()

                in_copy(c, b).wait()
                s0 = (s // 8) * 8
                off0 = block_off(s0, 0)
                for cp in stream_copies(off0):
                    cp.wait()
                cw_base = (cbase + c) * CHUNK_WORDS
                scatter_block(b, cw_base, off0, s, e)

                nblk = (e - s0 + K - 1) // K

                @pl.when(nblk > 1)
                def _():
                    def extra(b2, carry2):
                        offb = block_off(s0, b2)
                        for cp in stream_copies(offb):
                            cp.start()
                        for cp in stream_copies(offb):
                            cp.wait()
                        scatter_block(b, cw_base, offb, s, e)
                        return carry2

                    lax.fori_loop(1, nblk, extra, 0)

                s_next = _scalar_at(starts_v, cbase + c + 1)
                e_next = _scalar_at(starts_v, cbase + c + 2)

                @pl.when(c + 1 < cpt)
                def _():
                    for cp in stream_copies(block_off((s_next // 8) * 8, 0)):
                        cp.start()

                out_copy(c, b).start()
                s, e = s_next, e_next
            return (s, e)

        lax.fori_loop(0, cpt // 2, outer, (s0_first, e_first))
        out_copy(cpt - 1, 1).wait()

    return impl


def kernel(data, imps, rows, cols):
    n_rows, d = data.shape
    assert d == D and n_rows % (CHUNK_ROWS * NW) == 0
    n_imps = imps.shape[0]
    if 0 < n_imps < K:
        pad = K - n_imps
        imps = jnp.pad(imps, (0, pad))
        rows = jnp.pad(rows, (0, pad))
        cols = jnp.pad(cols, (0, pad))

    num_chunks = n_rows // CHUNK_ROWS
    bounds = jnp.arange(0, n_rows + 1, CHUNK_ROWS, dtype=jnp.int32)
    starts = jnp.searchsorted(rows[:n_imps], bounds, side="left").astype(jnp.int32)
    starts_pad = ((num_chunks + 2 + 7) // 8) * 8
    starts = jnp.concatenate(
        [starts, jnp.full((starts_pad - num_chunks - 1,), n_imps, jnp.int32)])

    impl = _make_impl(n_rows, n_imps, starts_pad)
    out = impl(data.reshape(-1), imps, rows, cols, starts)
    return out.reshape(n_rows, D)


# speedup vs baseline: 80.8438x; 1.4604x over previous
import functools

import jax
import jax.numpy as jnp
from jax import lax
from jax.experimental import pallas as pl
from jax.experimental.pallas import tpu as pltpu
from jax.experimental.pallas import tpu_sc as plsc

NC = 2
NS = 16
NW = NC * NS
L = 16

D = 64
W = 512
K = 12288
N_ROWS = 1_000_000
NCH = N_ROWS // W
TAIL0 = NCH * W
TAIL_N = N_ROWS - TAIL0
TAIL_WIN = 4224
NSLOT = NCH // NW + 1


def _scalar_at(buf, j):
    vec = plsc.load_gather(buf, [jnp.full((L,), j, jnp.int32)])
    return lax.reduce_max(vec, axes=(0,))


def _make_impl(n_imps, starts_pad):
    nlast = max(0, ((max(n_imps, K) - K + 7) // 8) * 8)
    mesh = plsc.VectorSubcoreMesh(core_axis_name="c", subcore_axis_name="s")

    @functools.partial(
        pl.kernel,
        out_type=jax.ShapeDtypeStruct((D, N_ROWS), jnp.float32),
        mesh=mesh,
        compiler_params=pltpu.CompilerParams(needs_layout_passes=False),
        scratch_types=[
            pltpu.VMEM((D, W), jnp.float32),
            pltpu.VMEM((D, W), jnp.float32),
            pltpu.VMEM((K,), jnp.int32),
            pltpu.VMEM((K,), jnp.int32),
            pltpu.VMEM((K,), jnp.float32),
            pltpu.VMEM((starts_pad,), jnp.int32),
            pltpu.SemaphoreType.DMA,
            pltpu.SemaphoreType.DMA,
            pltpu.SemaphoreType.DMA,
            pltpu.SemaphoreType.DMA,
            pltpu.SemaphoreType.DMA,
        ],
    )
    def impl(data_h, imps_h, rows_h, cols_h, starts_h, out_h,
             chunk_v0, chunk_v1, rows_v, cols_v, vals_v, starts_v,
             sem_in0, sem_in1, sem_out0, sem_out1, sem_st):
        chunks = (chunk_v0, chunk_v1)
        sems_in = (sem_in0, sem_in1)
        sems_out = (sem_out0, sem_out1)
        wid = lax.axis_index("s") * NC + lax.axis_index("c")
        pltpu.sync_copy(starts_h, starts_v)
        iota = lax.iota(jnp.int32, L)

        def in_copy(c, b):
            off = pl.multiple_of(c * W, 128)
            return pltpu.make_async_copy(
                data_h.at[:, pl.ds(off, W)], chunks[b], sems_in[b])

        def out_copy(c, b):
            off = pl.multiple_of(c * W, 128)
            return pltpu.make_async_copy(
                chunks[b], out_h.at[:, pl.ds(off, W)], sems_out[b])

        def stream_copies(off):
            off = pl.multiple_of(off, 8)
            return (
                pltpu.make_async_copy(rows_h.at[pl.ds(off, K)], rows_v, sem_st),
                pltpu.make_async_copy(cols_h.at[pl.ds(off, K)], cols_v, sem_st),
                pltpu.make_async_copy(imps_h.at[pl.ds(off, K)], vals_v, sem_st),
            )

        def block_off(s0, b2):
            return pl.multiple_of(jnp.minimum(s0 + b2 * K, nlast), 8)

        def scatter_block(ref, colbase, off, s, e):
            hi = jnp.minimum(e, off + K)
            ng = (hi - off + L - 1) // L
            glo = jnp.minimum((jnp.maximum(s - off, 0) + L - 1) // L, ng)
            ghi = jnp.maximum((hi - off) // L, glo)

            def masked(g, carry):
                sl = pl.ds(g * L, L)
                r = rows_v[sl]
                cc = cols_v[sl]
                v = vals_v[sl]
                p = off + g * L + iota
                m = (p >= s) & (p < e)
                plsc.store_scatter(ref, [cc, r - colbase], v, mask=m)
                return carry

            lax.fori_loop(0, glo, masked, 0)

            @plsc.parallel_loop(glo, ghi, unroll=4)
            def _(g):
                sl = pl.ds(g * L, L)
                r = rows_v[sl]
                cc = cols_v[sl]
                v = vals_v[sl]
                plsc.store_scatter(ref, [cc, r - colbase], v)

            lax.fori_loop(ghi, ng, masked, 0)

        def process(c, b, s, e):
            s0 = (s // 8) * 8
            off0 = block_off(s0, 0)
            for cp in stream_copies(off0):
                cp.wait()
            scatter_block(chunks[b], c * W, off0, s, e)

            nblk = (e - s0 + K - 1) // K

            @pl.when(nblk > 1)
            def _():
                def extra(b2, carry2):
                    offb = block_off(s0, b2)
                    for cp in stream_copies(offb):
                        cp.start()
                    for cp in stream_copies(offb):
                        cp.wait()
                    scatter_block(chunks[b], c * W, offb, s, e)
                    return carry2

                lax.fori_loop(1, nblk, extra, 0)

        s_first = _scalar_at(starts_v, wid)
        e_first = _scalar_at(starts_v, wid + 1)
        in_copy(wid, 0).start()
        for cp in stream_copies(block_off((s_first // 8) * 8, 0)):
            cp.start()

        def outer(jo, carry):
            s, e = carry
            for b in (0, 1):
                k = 2 * jo + b
                c = wid + NW * k
                nb = 1 - b

                @pl.when((c < NCH) & (k >= 1))
                def _():
                    out_copy(c - NW, nb).wait()

                @pl.when(c + NW < NCH)
                def _():
                    in_copy(c + NW, nb).start()

                @pl.when(c < NCH)
                def _():
                    in_copy(c, b).wait()
                    process(c, b, s, e)

                s_next = _scalar_at(starts_v, jnp.minimum(c + NW, NCH))
                e_next = _scalar_at(starts_v, jnp.minimum(c + NW + 1, NCH))

                @pl.when(c + NW < NCH)
                def _():
                    for cp in stream_copies(block_off((s_next // 8) * 8, 0)):
                        cp.start()

                @pl.when(c < NCH)
                def _():
                    out_copy(c, b).start()

                s, e = s_next, e_next
            return (s, e)

        lax.fori_loop(0, NSLOT // 2, outer, (s_first, e_first))

        @pl.when(wid == 0)
        def _():
            out_copy(NW * (NSLOT - 1), 1).wait()

        @pl.when(wid > 0)
        def _():
            out_copy(wid + NW * (NSLOT - 2), 0).wait()

    return impl


def _tail_fixup(data, imps, rows, cols, n_imps):
    tail_data = data[TAIL0:]
    m = min(TAIL_WIN, n_imps)
    if m == 0:
        return tail_data
    rr = rows[-m:] - TAIL0
    cc = cols[-m:]
    vv = imps[-m:]
    ridx = jnp.arange(TAIL_N, dtype=jnp.int32)
    cidx = jnp.arange(D, dtype=jnp.int32)
    r1h = (rr[:, None] == ridx[None, :]).astype(jnp.int32)
    c1h = (cc[:, None] == cidx[None, :]).astype(jnp.int32)
    vv_bits = lax.bitcast_convert_type(vv, jnp.int32)
    upd_bits = (r1h * vv_bits[:, None]).T @ c1h
    upd = lax.bitcast_convert_type(upd_bits, jnp.float32)
    hit = r1h.T @ c1h
    return jnp.where(hit > 0, upd, tail_data)


def kernel(data, imps, rows, cols):
    n_rows, d = data.shape
    assert d == D and n_rows == N_ROWS
    n_imps = imps.shape[0]
    if 0 < n_imps < K:
        pad = K - n_imps
        imps_p = jnp.pad(imps, (0, pad))
        rows_p = jnp.pad(rows, (0, pad))
        cols_p = jnp.pad(cols, (0, pad))
    else:
        imps_p, rows_p, cols_p = imps, rows, cols

    bounds = jnp.arange(0, TAIL0 + 1, W, dtype=jnp.int32)
    starts = jnp.searchsorted(rows_p[:n_imps], bounds, side="left").astype(jnp.int32)
    starts_pad = ((NCH + 1 + 7) // 8) * 8
    starts = jnp.concatenate(
        [starts, jnp.full((starts_pad - NCH - 1,), n_imps, jnp.int32)])

    impl = _make_impl(n_imps, starts_pad)
    out = impl(data.T, imps_p, rows_p, cols_p, starts).T
    tail = _tail_fixup(data, imps, rows, cols, n_imps)
    return lax.dynamic_update_slice(out, tail, (TAIL0, 0))


# speedup vs baseline: 80.9550x; 1.0014x over previous
import functools

import jax
import jax.numpy as jnp
from jax import lax
from jax.experimental import pallas as pl
from jax.experimental.pallas import tpu as pltpu
from jax.experimental.pallas import tpu_sc as plsc

NC = 2
NS = 16
NW = NC * NS
L = 16

D = 64
W = 512
K = 12288
N_ROWS = 1_000_000
NCH = N_ROWS // W
TAIL0 = NCH * W
TAIL_N = N_ROWS - TAIL0
TAIL_WIN = 4224
NSLOT = NCH // NW + 1


def _scalar_at(buf, j):
    vec = plsc.load_gather(buf, [jnp.full((L,), j, jnp.int32)])
    return lax.reduce_max(vec, axes=(0,))


def _make_impl(n_imps, starts_pad):
    nlast = max(0, ((max(n_imps, K) - K + 7) // 8) * 8)
    mesh = plsc.VectorSubcoreMesh(core_axis_name="c", subcore_axis_name="s")

    @functools.partial(
        pl.kernel,
        out_type=jax.ShapeDtypeStruct((D, N_ROWS), jnp.float32),
        mesh=mesh,
        compiler_params=pltpu.CompilerParams(needs_layout_passes=False),
        scratch_types=[
            pltpu.VMEM((D, W), jnp.float32),
            pltpu.VMEM((D, W), jnp.float32),
            pltpu.VMEM((K,), jnp.int32),
            pltpu.VMEM((K,), jnp.int32),
            pltpu.VMEM((K,), jnp.float32),
            pltpu.VMEM((starts_pad,), jnp.int32),
            pltpu.SemaphoreType.DMA,
            pltpu.SemaphoreType.DMA,
            pltpu.SemaphoreType.DMA,
            pltpu.SemaphoreType.DMA,
            pltpu.SemaphoreType.DMA,
        ],
    )
    def impl(data_h, imps_h, rows_h, cols_h, starts_h, out_h,
             chunk_v0, chunk_v1, rows_v, cols_v, vals_v, starts_v,
             sem_in0, sem_in1, sem_out0, sem_out1, sem_st):
        chunks = (chunk_v0, chunk_v1)
        sems_in = (sem_in0, sem_in1)
        sems_out = (sem_out0, sem_out1)
        wid = lax.axis_index("s") * NC + lax.axis_index("c")
        pltpu.sync_copy(starts_h, starts_v)
        iota = lax.iota(jnp.int32, L)

        def in_copy(c, b):
            off = pl.multiple_of(c * W, 128)
            return pltpu.make_async_copy(
                data_h.at[:, pl.ds(off, W)], chunks[b], sems_in[b])

        def out_copy(c, b):
            off = pl.multiple_of(c * W, 128)
            return pltpu.make_async_copy(
                chunks[b], out_h.at[:, pl.ds(off, W)], sems_out[b])

        def stream_copies(off):
            off = pl.multiple_of(off, 8)
            return (
                pltpu.make_async_copy(rows_h.at[pl.ds(off, K)], rows_v, sem_st),
                pltpu.make_async_copy(cols_h.at[pl.ds(off, K)], cols_v, sem_st),
                pltpu.make_async_copy(imps_h.at[pl.ds(off, K)], vals_v, sem_st),
            )

        def block_off(s0, b2):
            return pl.multiple_of(jnp.minimum(s0 + b2 * K, nlast), 8)

        def scatter_block(ref, colbase, off, s, e):
            hi = jnp.minimum(e, off + K)
            ng = (hi - off + L - 1) // L
            glo = jnp.minimum((jnp.maximum(s - off, 0) + L - 1) // L, ng)
            ghi = jnp.maximum((hi - off) // L, glo)

            def masked(g, carry):
                sl = pl.ds(g * L, L)
                r = rows_v[sl]
                cc = cols_v[sl]
                v = vals_v[sl]
                p = off + g * L + iota
                m = (p >= s) & (p < e)
                plsc.store_scatter(ref, [cc, r - colbase], v, mask=m)
                return carry

            lax.fori_loop(0, glo, masked, 0)

            @plsc.parallel_loop(glo, ghi, unroll=4)
            def _(g):
                sl = pl.ds(g * L, L)
                r = rows_v[sl]
                cc = cols_v[sl]
                v = vals_v[sl]
                plsc.store_scatter(ref, [cc, r - colbase], v)

            lax.fori_loop(ghi, ng, masked, 0)

        def process(c, b, s, e):
            s0 = (s // 8) * 8
            off0 = block_off(s0, 0)
            for cp in stream_copies(off0):
                cp.wait()
            scatter_block(chunks[b], c * W, off0, s, e)

            nblk = (e - s0 + K - 1) // K

            @pl.when(nblk > 1)
            def _():
                def extra(b2, carry2):
                    offb = block_off(s0, b2)
                    for cp in stream_copies(offb):
                        cp.start()
                    for cp in stream_copies(offb):
                        cp.wait()
                    scatter_block(chunks[b], c * W, offb, s, e)
                    return carry2

                lax.fori_loop(1, nblk, extra, 0)

        s_first = _scalar_at(starts_v, wid)
        e_first = _scalar_at(starts_v, wid + 1)
        in_copy(wid, 0).start()
        for cp in stream_copies(block_off((s_first // 8) * 8, 0)):
            cp.start()

        def outer(jo, carry):
            s, e = carry
            for b in (0, 1):
                k = 2 * jo + b
                c = wid + NW * k
                nb = 1 - b

                @pl.when((c < NCH) & (k >= 1))
                def _():
                    out_copy(c - NW, nb).wait()

                @pl.when(c + NW < NCH)
                def _():
                    in_copy(c + NW, nb).start()

                @pl.when(c < NCH)
                def _():
                    in_copy(c, b).wait()
                    process(c, b, s, e)

                s_next = _scalar_at(starts_v, jnp.minimum(c + NW, NCH))
                e_next = _scalar_at(starts_v, jnp.minimum(c + NW + 1, NCH))

                @pl.when(c + NW < NCH)
                def _():
                    for cp in stream_copies(block_off((s_next // 8) * 8, 0)):
                        cp.start()

                @pl.when(c < NCH)
                def _():
                    out_copy(c, b).start()

                s, e = s_next, e_next
            return (s, e)

        lax.fori_loop(0, NSLOT // 2, outer, (s_first, e_first))

        @pl.when(wid == 0)
        def _():
            out_copy(NW * (NSLOT - 1), 1).wait()

        @pl.when(wid > 0)
        def _():
            out_copy(wid + NW * (NSLOT - 2), 0).wait()

    return impl


def _tail_fixup(data, imps, rows, cols, n_imps):
    tail_data = data[TAIL0:]
    m = min(TAIL_WIN, n_imps)
    if m == 0:
        return tail_data
    rr = rows[-m:] - TAIL0
    cc = cols[-m:]
    vv = imps[-m:]
    ridx = jnp.arange(TAIL_N, dtype=jnp.int32)
    cidx = jnp.arange(D, dtype=jnp.int32)
    r1h = (rr[:, None] == ridx[None, :]).astype(jnp.int32)
    c1h = (cc[:, None] == cidx[None, :]).astype(jnp.int32)
    vv_bits = lax.bitcast_convert_type(vv, jnp.int32)
    upd_bits = (r1h * vv_bits[:, None]).T @ c1h
    upd = lax.bitcast_convert_type(upd_bits, jnp.float32)
    hit = r1h.T @ c1h
    return jnp.where(hit > 0, upd, tail_data)


def kernel(data, imps, rows, cols):
    n_rows, d = data.shape
    assert d == D and n_rows == N_ROWS
    n_imps = imps.shape[0]
    if 0 < n_imps < K:
        pad = K - n_imps
        imps_p = jnp.pad(imps, (0, pad))
        rows_p = jnp.pad(rows, (0, pad))
        cols_p = jnp.pad(cols, (0, pad))
    else:
        imps_p, rows_p, cols_p = imps, rows, cols

    bounds = jnp.arange(0, TAIL0 + 1, W, dtype=jnp.int32)
    starts = jnp.searchsorted(
        rows_p[:n_imps], bounds, side="left", method="scan_unrolled"
    ).astype(jnp.int32)
    starts_pad = ((NCH + 1 + 7) // 8) * 8
    starts = jnp.concatenate(
        [starts, jnp.full((starts_pad - NCH - 1,), n_imps, jnp.int32)])

    impl = _make_impl(n_imps, starts_pad)
    out = impl(data.T, imps_p, rows_p, cols_p, starts).T
    tail = _tail_fixup(data, imps, rows, cols, n_imps)
    return lax.dynamic_update_slice(out, tail, (TAIL0, 0))


# speedup vs baseline: 270.5687x; 3.3422x over previous
import functools

import jax
import jax.numpy as jnp
from jax import lax
from jax.experimental import pallas as pl
from jax.experimental.pallas import tpu as pltpu
from jax.experimental.pallas import tpu_sc as plsc

NC = 2
NS = 16
NW = NC * NS
L = 16

D = 64
W = 512
K = 12288
N_ROWS = 1_000_000
NCH = N_ROWS // W
TAIL0 = NCH * W
TAIL_N = N_ROWS - TAIL0
TAIL_WIN = 4224
NSLOT = NCH // NW + 1
NB = 2 * NSLOT
NBP = 128


def _scalar_at(buf, j):
    vec = plsc.load_gather(buf, [jnp.full((L,), j, jnp.int32)])
    return lax.reduce_max(vec, axes=(0,))


def _make_impl(n_imps):
    nlast = max(0, ((max(n_imps, K) - K + 7) // 8) * 8)
    has_imps = n_imps > 0
    rounds = max(1, int(n_imps).bit_length())
    mesh = plsc.VectorSubcoreMesh(core_axis_name="c", subcore_axis_name="s")

    @functools.partial(
        pl.kernel,
        out_type=jax.ShapeDtypeStruct((D, N_ROWS), jnp.float32),
        mesh=mesh,
        compiler_params=pltpu.CompilerParams(needs_layout_passes=False),
        scratch_types=[
            pltpu.VMEM((D, W), jnp.float32),
            pltpu.VMEM((D, W), jnp.float32),
            pltpu.VMEM((K,), jnp.int32),
            pltpu.VMEM((K,), jnp.int32),
            pltpu.VMEM((K,), jnp.float32),
            pltpu.VMEM((NBP,), jnp.int32),
            pltpu.VMEM((NBP,), jnp.int32),
            pltpu.VMEM((NBP,), jnp.int32),
            pltpu.VMEM((NBP,), jnp.int32),
            pltpu.VMEM((NBP,), jnp.int32),
            pltpu.SemaphoreType.DMA,
            pltpu.SemaphoreType.DMA,
            pltpu.SemaphoreType.DMA,
            pltpu.SemaphoreType.DMA,
            pltpu.SemaphoreType.DMA,
        ],
    )
    def impl(data_h, imps_h, rows_h, cols_h, out_h,
             chunk_v0, chunk_v1, rows_v, cols_v, vals_v,
             tgt_v, lo_v, hi_v, mid_v, gat_v,
             sem_in0, sem_in1, sem_out0, sem_out1, sem_st):
        chunks = (chunk_v0, chunk_v1)
        sems_in = (sem_in0, sem_in1)
        sems_out = (sem_out0, sem_out1)
        wid = lax.axis_index("s") * NC + lax.axis_index("c")
        iota = lax.iota(jnp.int32, L)

        for g in range(NBP // L):
            idx = g * L + iota
            j = jnp.minimum(wid + (idx >> 1) * NW + (idx & 1), NCH)
            tgt_v[pl.ds(g * L, L)] = j * W
            lo_v[pl.ds(g * L, L)] = jnp.zeros((L,), jnp.int32)
            hi_v[pl.ds(g * L, L)] = jnp.full((L,), n_imps, jnp.int32)

        if has_imps:
            def search_round(_, carry):
                for g in range(NBP // L):
                    sl = pl.ds(g * L, L)
                    lo = lo_v[sl]
                    hi = hi_v[sl]
                    mid = (lo + hi) >> 1
                    mid_v[sl] = jnp.minimum(mid, n_imps - 1)
                pltpu.async_copy(rows_h.at[mid_v], gat_v, sem_st).wait()
                for g in range(NBP // L):
                    sl = pl.ds(g * L, L)
                    lo = lo_v[sl]
                    hi = hi_v[sl]
                    mid = (lo + hi) >> 1
                    active = lo < hi
                    less = gat_v[sl] < tgt_v[sl]
                    lo_v[sl] = jnp.where(active & less, mid + 1, lo)
                    hi_v[sl] = jnp.where(active & (~less), mid, hi)
                return carry

            lax.fori_loop(0, rounds, search_round, 0)

        def in_copy(c, b):
            off = pl.multiple_of(c * W, 128)
            return pltpu.make_async_copy(
                data_h.at[:, pl.ds(off, W)], chunks[b], sems_in[b])

        def out_copy(c, b):
            off = pl.multiple_of(c * W, 128)
            return pltpu.make_async_copy(
                chunks[b], out_h.at[:, pl.ds(off, W)], sems_out[b])

        def stream_copies(off):
            off = pl.multiple_of(off, 8)
            return (
                pltpu.make_async_copy(rows_h.at[pl.ds(off, K)], rows_v, sem_st),
                pltpu.make_async_copy(cols_h.at[pl.ds(off, K)], cols_v, sem_st),
                pltpu.make_async_copy(imps_h.at[pl.ds(off, K)], vals_v, sem_st),
            )

        def block_off(s0, b2):
            return pl.multiple_of(jnp.minimum(s0 + b2 * K, nlast), 8)

        def scatter_block(ref, colbase, off, s, e):
            hi = jnp.minimum(e, off + K)
            ng = (hi - off + L - 1) // L
            glo = jnp.minimum((jnp.maximum(s - off, 0) + L - 1) // L, ng)
            ghi = jnp.maximum((hi - off) // L, glo)

            def masked(g, carry):
                sl = pl.ds(g * L, L)
                r = rows_v[sl]
                cc = cols_v[sl]
                v = vals_v[sl]
                p = off + g * L + iota
                m = (p >= s) & (p < e)
                plsc.store_scatter(ref, [cc, r - colbase], v, mask=m)
                return carry

            lax.fori_loop(0, glo, masked, 0)

            @plsc.parallel_loop(glo, ghi, unroll=4)
            def _(g):
                sl = pl.ds(g * L, L)
                r = rows_v[sl]
                cc = cols_v[sl]
                v = vals_v[sl]
                plsc.store_scatter(ref, [cc, r - colbase], v)

            lax.fori_loop(ghi, ng, masked, 0)

        def process(c, b, s, e):
            s0 = (s // 8) * 8
            off0 = block_off(s0, 0)
            for cp in stream_copies(off0):
                cp.wait()
            scatter_block(chunks[b], c * W, off0, s, e)

            nblk = (e - s0 + K - 1) // K

            @pl.when(nblk > 1)
            def _():
                def extra(b2, carry2):
                    offb = block_off(s0, b2)
                    for cp in stream_copies(offb):
                        cp.start()
                    for cp in stream_copies(offb):
                        cp.wait()
                    scatter_block(chunks[b], c * W, offb, s, e)
                    return carry2

                lax.fori_loop(1, nblk, extra, 0)

        s_first = _scalar_at(lo_v, 0)
        e_first = _scalar_at(lo_v, 1)
        in_copy(wid, 0).start()
        if has_imps:
            for cp in stream_copies(block_off((s_first // 8) * 8, 0)):
                cp.start()

        def outer(jo, carry):
            s, e = carry
            for b in (0, 1):
                k = 2 * jo + b
                c = wid + NW * k
                nb = 1 - b

                @pl.when((c < NCH) & (k >= 1))
                def _():
                    out_copy(c - NW, nb).wait()

                @pl.when(c + NW < NCH)
                def _():
                    in_copy(c + NW, nb).start()

                @pl.when(c < NCH)
                def _():
                    in_copy(c, b).wait()
                    if has_imps:
                        process(c, b, s, e)

                nxt = jnp.minimum(2 * k + 2, NBP - 2)
                s_next = _scalar_at(lo_v, nxt)
                e_next = _scalar_at(lo_v, nxt + 1)

                if has_imps:
                    @pl.when(c + NW < NCH)
                    def _():
                        for cp in stream_copies(block_off((s_next // 8) * 8, 0)):
                            cp.start()

                @pl.when(c < NCH)
                def _():
                    out_copy(c, b).start()

                s, e = s_next, e_next
            return (s, e)

        lax.fori_loop(0, NSLOT // 2, outer, (s_first, e_first))

        @pl.when(wid == 0)
        def _():
            out_copy(NW * (NSLOT - 1), 1).wait()

        @pl.when(wid > 0)
        def _():
            out_copy(wid + NW * (NSLOT - 2), 0).wait()

    return impl


def _tail_fixup(data, imps, rows, cols, n_imps):
    tail_data = data[TAIL0:]
    m = min(TAIL_WIN, n_imps)
    if m == 0:
        return tail_data
    rr = rows[-m:] - TAIL0
    cc = cols[-m:]
    vv = imps[-m:]
    ridx = jnp.arange(TAIL_N, dtype=jnp.int32)
    cidx = jnp.arange(D, dtype=jnp.int32)
    r1h = (rr[:, None] == ridx[None, :]).astype(jnp.int32)
    c1h = (cc[:, None] == cidx[None, :]).astype(jnp.int32)
    vv_bits = lax.bitcast_convert_type(vv, jnp.int32)
    upd_bits = (r1h * vv_bits[:, None]).T @ c1h
    upd = lax.bitcast_convert_type(upd_bits, jnp.float32)
    hit = r1h.T @ c1h
    return jnp.where(hit > 0, upd, tail_data)


def kernel(data, imps, rows, cols):
    n_rows, d = data.shape
    assert d == D and n_rows == N_ROWS
    n_imps = imps.shape[0]
    if 0 < n_imps < K:
        pad = K - n_imps
        imps_p = jnp.pad(imps, (0, pad))
        rows_p = jnp.pad(rows, (0, pad))
        cols_p = jnp.pad(cols, (0, pad))
    else:
        imps_p, rows_p, cols_p = imps, rows, cols

    impl = _make_impl(n_imps)
    out = impl(data.T, imps_p, rows_p, cols_p).T
    tail = _tail_fixup(data, imps, rows, cols, n_imps)
    return lax.dynamic_update_slice(out, tail, (TAIL0, 0))
